# Initial kernel scaffold; baseline (speedup 1.0000x reference)
import jax
import jax.numpy as jnp
from jax.experimental import pallas as pl


def kernel(x, gate_W, gate_b, expert_W, expert_b):
    raise NotImplementedError("write your pallas kernel here")


# speedup vs baseline: 1.0897x; 1.0897x over previous
import jax
import jax.numpy as jnp
from jax.experimental import pallas as pl
from jax.experimental.pallas import tpu as pltpu

_T = 8192
_D = 2048
_E = 8
_BT = 512
_BN = 256
_LANES = 128
_NEG = -1e30


def _gate_body(x_ref, gw_ref, gb_ref, cw_ref, sc_ref):
    x = x_ref[...]
    logits = jnp.dot(x, gw_ref[...], preferred_element_type=jnp.float32)
    logits = logits + gb_ref[...]
    m = jnp.max(logits, axis=1, keepdims=True)
    p = jnp.exp(logits - m)
    scores = p / jnp.sum(p, axis=1, keepdims=True)
    lane = jax.lax.broadcasted_iota(jnp.int32, scores.shape, 1)
    m1 = jnp.max(scores, axis=1, keepdims=True)
    e0 = jnp.min(jnp.where(scores == m1, lane, _LANES), axis=1, keepdims=True)
    sc2 = jnp.where(lane == e0, -1.0, scores)
    m2 = jnp.max(sc2, axis=1, keepdims=True)
    e1 = jnp.min(jnp.where(sc2 == m2, lane, _LANES), axis=1, keepdims=True)
    keep = (lane == e0) | (lane == e1)
    cw_ref[...] = jnp.where(keep, scores, 0.0)
    sc_ref[...] = scores


def _moe_body(cw_ref, sc_ref, x_ref, w_ref, bp_ref, o_ref):
    sc = sc_ref[...]
    acc = jnp.dot(sc, bp_ref[...], preferred_element_type=jnp.float32)
    x = x_ref[...]
    cw = cw_ref[...]
    for e in range(_E):
        pe = jnp.dot(x, w_ref[e, :, :], preferred_element_type=jnp.float32)
        acc = acc + pe * cw[:, e][:, None]
    o_ref[...] = acc


def kernel(x, gate_W, gate_b, expert_W, expert_b):
    x_bf = x.astype(jnp.bfloat16)
    w_bf = expert_W.astype(jnp.bfloat16)
    gw_pad = jnp.zeros((_D, _LANES), jnp.float32).at[:, :_E].set(gate_W)
    gb_pad = jnp.full((1, _LANES), _NEG, jnp.float32).at[0, :_E].set(gate_b)
    bp = jnp.zeros((_LANES, _D), jnp.float32).at[:_E].set(expert_b)

    n_tb = _T // _BT
    cw, sc = pl.pallas_call(
        _gate_body,
        grid=(n_tb,),
        in_specs=[
            pl.BlockSpec((_BT, _D), lambda i: (i, 0)),
            pl.BlockSpec((_D, _LANES), lambda i: (0, 0)),
            pl.BlockSpec((1, _LANES), lambda i: (0, 0)),
        ],
        out_specs=[
            pl.BlockSpec((_BT, _LANES), lambda i: (i, 0)),
            pl.BlockSpec((_BT, _LANES), lambda i: (i, 0)),
        ],
        out_shape=[
            jax.ShapeDtypeStruct((_T, _LANES), jnp.float32),
            jax.ShapeDtypeStruct((_T, _LANES), jnp.float32),
        ],
    )(x, gw_pad, gb_pad)

    n_nb = _D // _BN
    out = pl.pallas_call(
        _moe_body,
        grid=(n_nb, n_tb),
        in_specs=[
            pl.BlockSpec((_BT, _LANES), lambda n, t: (t, 0)),
            pl.BlockSpec((_BT, _LANES), lambda n, t: (t, 0)),
            pl.BlockSpec((_BT, _D), lambda n, t: (t, 0)),
            pl.BlockSpec((_E, _D, _BN), lambda n, t: (0, 0, n)),
            pl.BlockSpec((_LANES, _BN), lambda n, t: (0, n)),
        ],
        out_specs=pl.BlockSpec((_BT, _BN), lambda n, t: (t, n)),
        out_shape=jax.ShapeDtypeStruct((_T, _D), jnp.float32),
    )(cw, sc, x_bf, w_bf, bp)
    return out
